# baseline (speedup 1.0000x reference)
import jax
import jax.numpy as jnp
from jax import lax
from jax.experimental import pallas as pl
from jax.experimental.pallas import tpu as pltpu
from jax.experimental.pallas import tpu_sc as plsc

NUM_SPARSE = 26
HIST_LEN = 50
VOCAB = 1000000
BATCH = 16384
NUM_FIELDS = NUM_SPARSE + HIST_LEN
NUM_TABLES = NUM_SPARSE + 1

L = 16
NW = 32
B_PER_W = BATCH // NW


def _logit_kernel(inputs_t_hbm, tables_hbm, out_hbm,
                  idx_t, vals, outbuf, sem):
    wid = lax.axis_index("s") * 2 + lax.axis_index("c")
    base = wid * B_PER_W

    pltpu.sync_copy(inputs_t_hbm.at[:, pl.ds(base, B_PER_W)], idx_t)

    copies = []
    for c in range(NUM_FIELDS):
        t = min(c, NUM_SPARSE)
        copies.append(pltpu.async_copy(
            tables_hbm.at[t].at[idx_t.at[c]],
            vals.at[c],
            sem))
    for cp in copies:
        cp.wait()

    def rbody(v, carry):
        o = v * L
        acc = jnp.zeros((L,), jnp.float32)
        for c in range(NUM_SPARSE):
            acc = acc + vals[c, pl.ds(o, L)]
        for c in range(NUM_SPARSE, NUM_FIELDS):
            val = vals[c, pl.ds(o, L)]
            raw = idx_t[c, pl.ds(o, L)]
            acc = acc + jnp.where(raw != 0, val, 0.0)
        outbuf[pl.ds(o, L)] = acc
        return carry

    lax.fori_loop(0, B_PER_W // L, rbody, 0)

    pltpu.sync_copy(outbuf, out_hbm.at[pl.ds(base, B_PER_W)])


@jax.jit
def _run(inputs_t, tables_2d):
    mesh = plsc.VectorSubcoreMesh(core_axis_name="c", subcore_axis_name="s")
    return pl.kernel(
        _logit_kernel,
        mesh=mesh,
        compiler_params=pltpu.CompilerParams(
            needs_layout_passes=False, use_tc_tiling_on_sc=False),
        out_type=jax.ShapeDtypeStruct((BATCH,), jnp.float32),
        scratch_types=[
            pltpu.VMEM((NUM_FIELDS, B_PER_W), jnp.int32),
            pltpu.VMEM((NUM_FIELDS, B_PER_W), jnp.float32),
            pltpu.VMEM((B_PER_W,), jnp.float32),
            pltpu.SemaphoreType.DMA,
        ],
    )(inputs_t, tables_2d)


def kernel(inputs, tables):
    return _run(inputs.T, tables[:, :, 0])


# speedup vs baseline: 4.5971x; 4.5971x over previous
import jax
import jax.numpy as jnp
from jax import lax
from jax.experimental import pallas as pl
from jax.experimental.pallas import tpu as pltpu
from jax.experimental.pallas import tpu_sc as plsc

NUM_SPARSE = 26
HIST_LEN = 50
VOCAB = 1000000
BATCH = 16384
NUM_FIELDS = NUM_SPARSE + HIST_LEN
NUM_TABLES = NUM_SPARSE + 1

L = 16
NW = 32
B_PER_W = BATCH // NW


def _logit_kernel(*refs):
    inputs_t_hbm = refs[0]
    table_refs = refs[1:1 + NUM_TABLES]
    out_hbm = refs[1 + NUM_TABLES]
    idx_t, vals, outbuf, sem = refs[2 + NUM_TABLES:]

    wid = lax.axis_index("s") * 2 + lax.axis_index("c")
    base = wid * B_PER_W

    pltpu.sync_copy(inputs_t_hbm.at[:, pl.ds(base, B_PER_W)], idx_t)

    copies = []
    for c in range(NUM_FIELDS):
        t = min(c, NUM_SPARSE)
        copies.append(pltpu.async_copy(
            table_refs[t].at[idx_t.at[c]],
            vals.at[c],
            sem))
    for cp in copies:
        cp.wait()

    def rbody(v, carry):
        o = v * L
        acc = jnp.zeros((L,), jnp.float32)
        for c in range(NUM_SPARSE):
            acc = acc + vals[c, pl.ds(o, L)]
        for c in range(NUM_SPARSE, NUM_FIELDS):
            val = vals[c, pl.ds(o, L)]
            raw = idx_t[c, pl.ds(o, L)]
            acc = acc + jnp.where(raw != 0, val, 0.0)
        outbuf[pl.ds(o, L)] = acc
        return carry

    lax.fori_loop(0, B_PER_W // L, rbody, 0)

    pltpu.sync_copy(outbuf, out_hbm.at[pl.ds(base, B_PER_W)])


@jax.jit
def _run(inputs_t, *tables_1d):
    mesh = plsc.VectorSubcoreMesh(core_axis_name="c", subcore_axis_name="s")
    return pl.kernel(
        _logit_kernel,
        mesh=mesh,
        compiler_params=pltpu.CompilerParams(
            needs_layout_passes=False, use_tc_tiling_on_sc=False),
        out_type=jax.ShapeDtypeStruct((BATCH,), jnp.float32),
        scratch_types=[
            pltpu.VMEM((NUM_FIELDS, B_PER_W), jnp.int32),
            pltpu.VMEM((NUM_FIELDS, B_PER_W), jnp.float32),
            pltpu.VMEM((B_PER_W,), jnp.float32),
            pltpu.SemaphoreType.DMA,
        ],
    )(inputs_t, *tables_1d)


def kernel(inputs, tables):
    tables_t = tables.transpose(0, 2, 1)
    tables_1d = tuple(tables_t[t, 0, :] for t in range(NUM_TABLES))
    return _run(inputs.T, *tables_1d)


# speedup vs baseline: 5.2459x; 1.1411x over previous
import jax
import jax.numpy as jnp
from jax import lax
from jax.experimental import pallas as pl
from jax.experimental.pallas import tpu as pltpu
from jax.experimental.pallas import tpu_sc as plsc

NUM_SPARSE = 26
HIST_LEN = 50
VOCAB = 1000000
BATCH = 16384
NUM_FIELDS = NUM_SPARSE + HIST_LEN
NUM_TABLES = NUM_SPARSE + 1

L = 16
NW = 32
B_PER_W = BATCH // NW

_CP = pltpu.CompilerParams(needs_layout_passes=False, use_tc_tiling_on_sc=False)
_MESH = dict(core_axis_name="c", subcore_axis_name="s")


def _hist_kernel(inputs_t_hbm, hist_hbm, out_hbm, idx_t, vals, outbuf, sem):
    wid = lax.axis_index("s") * 2 + lax.axis_index("c")
    base = wid * B_PER_W

    pltpu.sync_copy(
        inputs_t_hbm.at[pl.ds(NUM_SPARSE, HIST_LEN), pl.ds(base, B_PER_W)],
        idx_t)

    copies = []
    for c in range(HIST_LEN):
        copies.append(pltpu.async_copy(
            hist_hbm.at[idx_t.at[c]], vals.at[c], sem))
    for cp in copies:
        cp.wait()

    def rbody(v, carry):
        o = v * L
        acc = jnp.zeros((L,), jnp.float32)
        for c in range(HIST_LEN):
            val = vals[c, pl.ds(o, L)]
            raw = idx_t[c, pl.ds(o, L)]
            acc = acc + jnp.where(raw != 0, val, 0.0)
        outbuf[pl.ds(o, L)] = acc
        return carry

    lax.fori_loop(0, B_PER_W // L, rbody, 0)
    pltpu.sync_copy(outbuf, out_hbm.at[pl.ds(base, B_PER_W)])


def _sparse_kernel(*refs):
    inputs_t_hbm = refs[0]
    partial_hbm = refs[1]
    table_refs = refs[2:2 + NUM_SPARSE]
    out_hbm = refs[2 + NUM_SPARSE]
    idx_t, vals, outbuf, sem = refs[3 + NUM_SPARSE:]

    wid = lax.axis_index("s") * 2 + lax.axis_index("c")
    base = wid * B_PER_W

    pltpu.sync_copy(
        inputs_t_hbm.at[pl.ds(0, NUM_SPARSE), pl.ds(base, B_PER_W)], idx_t)
    pltpu.sync_copy(partial_hbm.at[pl.ds(base, B_PER_W)], outbuf)

    copies = []
    for c in range(NUM_SPARSE):
        copies.append(pltpu.async_copy(
            table_refs[c].at[idx_t.at[c]], vals.at[c], sem))
    for cp in copies:
        cp.wait()

    def rbody(v, carry):
        o = v * L
        acc = outbuf[pl.ds(o, L)]
        for c in range(NUM_SPARSE):
            acc = acc + vals[c, pl.ds(o, L)]
        outbuf[pl.ds(o, L)] = acc
        return carry

    lax.fori_loop(0, B_PER_W // L, rbody, 0)
    pltpu.sync_copy(outbuf, out_hbm.at[pl.ds(base, B_PER_W)])


@jax.jit
def _run(inputs_t, hist, *tables_1d):
    partial = pl.kernel(
        _hist_kernel,
        mesh=plsc.VectorSubcoreMesh(**_MESH),
        compiler_params=_CP,
        out_type=jax.ShapeDtypeStruct((BATCH,), jnp.float32),
        scratch_types=[
            pltpu.VMEM((HIST_LEN, B_PER_W), jnp.int32),
            pltpu.VMEM((HIST_LEN, B_PER_W), jnp.float32),
            pltpu.VMEM((B_PER_W,), jnp.float32),
            pltpu.SemaphoreType.DMA,
        ],
    )(inputs_t, hist)
    return pl.kernel(
        _sparse_kernel,
        mesh=plsc.VectorSubcoreMesh(**_MESH),
        compiler_params=_CP,
        out_type=jax.ShapeDtypeStruct((BATCH,), jnp.float32),
        scratch_types=[
            pltpu.VMEM((NUM_SPARSE, B_PER_W), jnp.int32),
            pltpu.VMEM((NUM_SPARSE, B_PER_W), jnp.float32),
            pltpu.VMEM((B_PER_W,), jnp.float32),
            pltpu.SemaphoreType.DMA,
        ],
    )(inputs_t, partial, *tables_1d)


def kernel(inputs, tables):
    hist = tables[NUM_SPARSE, :, 0]
    tables_1d = tuple(tables[t, :, 0] for t in range(NUM_SPARSE))
    return _run(inputs.T, hist, *tables_1d)


# speedup vs baseline: 5.3323x; 1.0165x over previous
import jax
import jax.numpy as jnp
from jax import lax
from jax.experimental import pallas as pl
from jax.experimental.pallas import tpu as pltpu
from jax.experimental.pallas import tpu_sc as plsc

NUM_SPARSE = 26
HIST_LEN = 50
VOCAB = 1000000
BATCH = 16384
NUM_FIELDS = NUM_SPARSE + HIST_LEN
NUM_TABLES = NUM_SPARSE + 1

L = 16
NW = 32
B_PER_W = BATCH // NW

_CP = pltpu.CompilerParams(needs_layout_passes=False, use_tc_tiling_on_sc=False)
_MESH = dict(core_axis_name="c", subcore_axis_name="s")


def _hist_kernel(inputs_t_hbm, hist_hbm, out_hbm, idx_t, vals, outbuf, sem):
    wid = lax.axis_index("s") * 2 + lax.axis_index("c")
    base = wid * B_PER_W

    pltpu.sync_copy(
        inputs_t_hbm.at[pl.ds(NUM_SPARSE, HIST_LEN), pl.ds(base, B_PER_W)],
        idx_t)

    copies = []
    for c in range(HIST_LEN):
        copies.append(pltpu.async_copy(
            hist_hbm.at[idx_t.at[c]], vals.at[c], sem))
    for cp in copies:
        cp.wait()

    def rbody(v, carry):
        o = v * L
        acc = jnp.zeros((L,), jnp.float32)
        for c in range(HIST_LEN):
            val = vals[c, pl.ds(o, L)]
            raw = idx_t[c, pl.ds(o, L)]
            acc = acc + jnp.where(raw != 0, val, 0.0)
        outbuf[pl.ds(o, L)] = acc
        return carry

    lax.fori_loop(0, B_PER_W // L, rbody, 0)
    pltpu.sync_copy(outbuf, out_hbm.at[pl.ds(base, B_PER_W)])


HALF = NUM_SPARSE // 2


def _make_sparse_kernel(col0, ncols):
    def _sparse_kernel(*refs):
        inputs_t_hbm = refs[0]
        partial_hbm = refs[1]
        table_refs = refs[2:2 + ncols]
        out_hbm = refs[2 + ncols]
        idx_t, vals, outbuf, sem = refs[3 + ncols:]

        wid = lax.axis_index("s") * 2 + lax.axis_index("c")
        base = wid * B_PER_W

        pltpu.sync_copy(
            inputs_t_hbm.at[pl.ds(col0, ncols), pl.ds(base, B_PER_W)], idx_t)
        pltpu.sync_copy(partial_hbm.at[pl.ds(base, B_PER_W)], outbuf)

        copies = []
        for c in range(ncols):
            copies.append(pltpu.async_copy(
                table_refs[c].at[idx_t.at[c]], vals.at[c], sem))
        for cp in copies:
            cp.wait()

        def rbody(v, carry):
            o = v * L
            acc = outbuf[pl.ds(o, L)]
            for c in range(ncols):
                acc = acc + vals[c, pl.ds(o, L)]
            outbuf[pl.ds(o, L)] = acc
            return carry

        lax.fori_loop(0, B_PER_W // L, rbody, 0)
        pltpu.sync_copy(outbuf, out_hbm.at[pl.ds(base, B_PER_W)])

    return _sparse_kernel


@jax.jit
def _run(inputs_t, hist, *tables_1d):
    partial = pl.kernel(
        _hist_kernel,
        mesh=plsc.VectorSubcoreMesh(**_MESH),
        compiler_params=_CP,
        out_type=jax.ShapeDtypeStruct((BATCH,), jnp.float32),
        scratch_types=[
            pltpu.VMEM((HIST_LEN, B_PER_W), jnp.int32),
            pltpu.VMEM((HIST_LEN, B_PER_W), jnp.float32),
            pltpu.VMEM((B_PER_W,), jnp.float32),
            pltpu.SemaphoreType.DMA,
        ],
    )(inputs_t, hist)
    for col0 in (0, HALF):
        partial = pl.kernel(
            _make_sparse_kernel(col0, HALF),
            mesh=plsc.VectorSubcoreMesh(**_MESH),
            compiler_params=_CP,
            out_type=jax.ShapeDtypeStruct((BATCH,), jnp.float32),
            scratch_types=[
                pltpu.VMEM((HALF, B_PER_W), jnp.int32),
                pltpu.VMEM((HALF, B_PER_W), jnp.float32),
                pltpu.VMEM((B_PER_W,), jnp.float32),
                pltpu.SemaphoreType.DMA,
            ],
        )(inputs_t, partial, *tables_1d[col0:col0 + HALF])
    return partial


def kernel(inputs, tables):
    hist = tables[NUM_SPARSE, :, 0]
    tables_1d = tuple(tables[t, :, 0] for t in range(NUM_SPARSE))
    return _run(inputs.T, hist, *tables_1d)
